# baseline (speedup 1.0000x reference)
import jax
import jax.numpy as jnp
from jax import lax
from jax.experimental import pallas as pl
from jax.experimental.pallas import tpu as pltpu
from jax.experimental.pallas import tpu_sc as plsc
import functools

NC = 2
NS = 16
L = 16

N = 10000
B = 8
NP = 1250
D = 128

E_REAL = 320000
EROWS = E_REAL // 128
CHE = 80
NB = 16
NBH = 16
BASE_ROWS = EROWS // 32

NPAD = 10240
HSL = NPAD // NS
GROW = 9
GP = GROW * NPAD
GSL = GP // NS

MAGIC = 6711
SHIFT = 23

_mesh = plsc.VectorSubcoreMesh(
    core_axis_name="c", subcore_axis_name="s", num_cores=NC, num_subcores=NS)
_sc_params = pltpu.CompilerParams(needs_layout_passes=False)


def _zero_vmem(ref, n):
    z = jnp.zeros((L,), jnp.float32)

    def body(i, _):
        ref[pl.ds(i * L, L)] = z
        return 0

    lax.fori_loop(0, n // L, body, 0)


def _fast_rsqrt(d):
    ii = lax.bitcast_convert_type(d, jnp.int32)
    ii = jnp.int32(0x5F3759DF) - (ii >> 1)
    y = lax.bitcast_convert_type(ii, jnp.float32)
    for _ in range(3):
        y = y * (1.5 - 0.5 * d * y * y)
    return y


def _tile_rows(tid):
    nrows = BASE_ROWS + (tid < 4).astype(jnp.int32)
    gb = BASE_ROWS * tid + jnp.minimum(tid, 4)
    gbs = jnp.minimum(gb, EROWS - CHE)
    shift = gb - gbs
    return nrows, gbs, shift


def _sanitize(ref, lo, hi):
    iota = lax.iota(jnp.int32, L)

    def body(j, _):
        for k in range(128 // L):
            ref[pl.ds(j * 128 + k * L, L)] = N + k * L + iota
        return 0

    lax.fori_loop(lo, hi, body, 0)


@functools.partial(
    pl.kernel,
    out_type=jax.ShapeDtypeStruct((NC * GP,), jnp.float32),
    mesh=_mesh,
    compiler_params=_sc_params,
    scratch_types=[
        pltpu.VMEM((CHE * 128,), jnp.int32),
        pltpu.VMEM((CHE * 128,), jnp.int32),
        pltpu.VMEM((CHE * 128,), jnp.int32),
        pltpu.VMEM((2 * NBH, 128), jnp.int32),
        pltpu.VMEM((128,), jnp.float32),
        pltpu.VMEM((HSL,), jnp.float32),
        pltpu.VMEM((HSL,), jnp.float32),
        pltpu.VMEM((HSL,), jnp.float32),
        pltpu.VMEM((HSL,), jnp.float32),
        pltpu.VMEM((NPAD,), jnp.float32),
        pltpu.VMEM((NPAD,), jnp.float32),
        pltpu.VMEM((GSL,), jnp.float32),
        pltpu.VMEM((NB, 128), jnp.float32),
        pltpu.VMEM((NB, 128), jnp.int32),
        pltpu.VMEM_SHARED((NPAD,), jnp.float32),
        pltpu.VMEM_SHARED((NPAD,), jnp.float32),
        pltpu.VMEM_SHARED((NPAD,), jnp.float32),
        pltpu.VMEM_SHARED((GP,), jnp.float32),
        pltpu.SemaphoreType.DMA,
    ],
)
def _k2_scatter(e_hbm, v_hbm, g_out,
                srcv, dstv, dstm, hrows, ones, hv, vslv, dslv, aslv,
                dinvv, av, zbuf,
                wrows, irows, hist_sh, dinv_sh, a_sh, g_sh, sem):
    c = lax.axis_index("c")
    s = lax.axis_index("s")
    tid = c * NS + s
    mtid = (1 - c) * NS + s

    for k in range(128 // L):
        ones[pl.ds(k * L, L)] = jnp.ones((L,), jnp.float32)
    _zero_vmem(zbuf, GSL)
    hoff = pl.multiple_of(s * HSL, 128)
    goff = pl.multiple_of(s * GSL, 128)
    pltpu.sync_copy(zbuf.at[pl.ds(0, HSL)], hist_sh.at[pl.ds(hoff, HSL)])
    pltpu.sync_copy(zbuf, g_sh.at[pl.ds(goff, GSL)])

    nrows, gbs, shift = _tile_rows(tid)
    mnrows, mgbs, mshift = _tile_rows(mtid)
    pltpu.sync_copy(e_hbm.at[0, pl.ds(gbs * 128, CHE * 128)], srcv)
    pltpu.sync_copy(e_hbm.at[1, pl.ds(gbs * 128, CHE * 128)], dstv)
    pltpu.sync_copy(e_hbm.at[1, pl.ds(mgbs * 128, CHE * 128)], dstm)
    _sanitize(srcv, 0, shift)
    _sanitize(srcv, shift + nrows, CHE)
    _sanitize(dstv, 0, shift)
    _sanitize(dstv, shift + nrows, CHE)
    _sanitize(dstm, 0, mshift)
    _sanitize(dstm, mshift + mnrows, CHE)
    plsc.subcore_barrier()

    def hblk(jb, _):
        descs = []
        for t in range(NBH):
            row = jb * NBH + t
            for k in range(128 // L):
                sl = pl.ds(k * L, L)
                hrows[t, sl] = dstv[pl.ds(row * 128 + k * L, L)]
                hrows[NBH + t, sl] = dstm[pl.ds(row * 128 + k * L, L)]
            descs.append(pltpu.async_copy(
                ones, hist_sh.at[hrows.at[t]], sem, add=True))
            descs.append(pltpu.async_copy(
                ones, hist_sh.at[hrows.at[NBH + t]], sem, add=True))
        for dsc in descs:
            dsc.wait()
        return 0

    lax.fori_loop(0, CHE // NBH, hblk, 0)
    plsc.subcore_barrier()

    pltpu.sync_copy(hist_sh.at[pl.ds(hoff, HSL)], hv)
    pltpu.sync_copy(v_hbm.at[pl.ds(hoff, HSL)], vslv)

    def dbody(i, _):
        sl = pl.ds(i * L, L)
        y = _fast_rsqrt(hv[sl] + 1.0)
        dslv[sl] = y
        aslv[sl] = vslv[sl] * y
        return 0

    lax.fori_loop(0, HSL // L, dbody, 0)
    pltpu.sync_copy(dslv, dinv_sh.at[pl.ds(hoff, HSL)])
    pltpu.sync_copy(aslv, a_sh.at[pl.ds(hoff, HSL)])
    plsc.subcore_barrier()
    pltpu.sync_copy(dinv_sh, dinvv)
    pltpu.sync_copy(a_sh, av)

    def _do_row(t, s16, d16):
        for k in range(128 // L):
            sl = pl.ds(k * L, L)
            wrows[t, sl] = plsc.load_gather(av, [d16[k]])
            irows[t, sl] = (d16[k] * MAGIC >> SHIFT) * NPAD + s16[k]
        return pltpu.async_copy(
            wrows.at[t], g_sh.at[irows.at[t]], sem, add=True)

    def blk(jb, _):
        descs = []
        for t in range(NB):
            row = jb * NB + t
            s16 = [srcv[pl.ds(row * 128 + k * L, L)] for k in range(128 // L)]
            d16 = [dstv[pl.ds(row * 128 + k * L, L)] for k in range(128 // L)]
            descs.append(_do_row(t, s16, d16))
        for dsc in descs:
            dsc.wait()
        return 0

    lax.fori_loop(0, CHE // NB, blk, 0)

    iota = lax.iota(jnp.int32, L)
    descs = []
    for t in range(NB):
        n16 = []
        for k in range(128 // L):
            if t < 3:
                n = (tid * 3 + t) * 128 + k * L + iota
                n16.append(jnp.minimum(n, NPAD - 1))
            else:
                n16.append(jnp.full((L,), NPAD - 1, jnp.int32))
        descs.append(_do_row(t, n16, n16))
    for dsc in descs:
        dsc.wait()

    plsc.subcore_barrier()

    pltpu.sync_copy(g_sh.at[pl.ds(goff, GSL)], zbuf)
    base_m = s * GSL - (s * GSL // NPAD) * NPAD

    def scbody(i, _):
        cm = base_m + i * L
        cm = cm - (cm >= NPAD).astype(jnp.int32) * NPAD
        sl = pl.ds(i * L, L)
        zbuf[sl] = zbuf[sl] * dinvv[pl.ds(cm, L)]
        return 0

    lax.fori_loop(0, GSL // L, scbody, 0)
    ooff = pl.multiple_of(c * GP + s * GSL, 128)
    pltpu.sync_copy(zbuf, g_out.at[pl.ds(ooff, GSL)])


def _k3_body(g_ref, x_ref, pm_ref, w_ref, b_ref, o_ref):
    gg = g_ref[...].reshape(NC * GROW, NPAD)
    g = (gg[0:B] + gg[GROW:GROW + B])[:, :N]
    gx = jnp.dot(g, x_ref[...], preferred_element_type=jnp.float32)
    r = jnp.dot(gx, w_ref[...], preferred_element_type=jnp.float32)
    msum = jnp.sum(pm_ref[...], axis=1)
    o_ref[...] = r + msum[:, None] * b_ref[...][None, :]


_k3_tail = pl.pallas_call(
    _k3_body,
    out_shape=jax.ShapeDtypeStruct((B, D), jnp.float32),
)


def kernel(x, edge_index, protein_mask, W, b):
    e = edge_index.astype(jnp.int32)
    v = jnp.concatenate(
        [protein_mask.reshape(-1), jnp.zeros((NPAD - N,), jnp.float32)])
    gflat = _k2_scatter(e, v)
    return _k3_tail(gflat, x, protein_mask, W, b)


# speedup vs baseline: 1.3444x; 1.3444x over previous
import jax
import jax.numpy as jnp
from jax import lax
from jax.experimental import pallas as pl
from jax.experimental.pallas import tpu as pltpu
from jax.experimental.pallas import tpu_sc as plsc
import functools

NC = 2
NS = 16
L = 16

N = 10000
B = 8
NP = 1250
D = 128

E_REAL = 320000
EROWS = E_REAL // 128
CHE = 80
NB = 8
NBH = 8
BASE_ROWS = EROWS // 32

NPAD = 10240
HSL = NPAD // NS
GROW = 9
GP = GROW * NPAD
GSL = GP // NS

MAGIC = 6711
SHIFT = 23

_mesh = plsc.VectorSubcoreMesh(
    core_axis_name="c", subcore_axis_name="s", num_cores=NC, num_subcores=NS)
_sc_params = pltpu.CompilerParams(needs_layout_passes=False)


def _zero_vmem(ref, n):
    z = jnp.zeros((L,), jnp.float32)

    def body(i, _):
        ref[pl.ds(i * L, L)] = z
        return 0

    lax.fori_loop(0, n // L, body, 0)


def _fast_rsqrt(d):
    ii = lax.bitcast_convert_type(d, jnp.int32)
    ii = jnp.int32(0x5F3759DF) - (ii >> 1)
    y = lax.bitcast_convert_type(ii, jnp.float32)
    for _ in range(3):
        y = y * (1.5 - 0.5 * d * y * y)
    return y


def _tile_rows(tid):
    nrows = BASE_ROWS + (tid < 4).astype(jnp.int32)
    gb = BASE_ROWS * tid + jnp.minimum(tid, 4)
    gbs = jnp.minimum(gb, EROWS - CHE)
    shift = gb - gbs
    return nrows, gbs, shift


def _sanitize(ref, lo, hi):
    iota = lax.iota(jnp.int32, L)

    def body(j, _):
        for k in range(128 // L):
            ref[pl.ds(j * 128 + k * L, L)] = N + k * L + iota
        return 0

    lax.fori_loop(lo, hi, body, 0)


@functools.partial(
    pl.kernel,
    out_type=jax.ShapeDtypeStruct((NC * GP,), jnp.float32),
    mesh=_mesh,
    compiler_params=_sc_params,
    scratch_types=[
        pltpu.VMEM((CHE * 128,), jnp.int32),
        pltpu.VMEM((CHE * 128,), jnp.int32),
        pltpu.VMEM((CHE * 128,), jnp.int32),
        pltpu.VMEM((2 * NBH, 128), jnp.int32),
        pltpu.VMEM((128,), jnp.float32),
        pltpu.VMEM((HSL,), jnp.float32),
        pltpu.VMEM((HSL,), jnp.float32),
        pltpu.VMEM((HSL,), jnp.float32),
        pltpu.VMEM((HSL,), jnp.float32),
        pltpu.VMEM((NPAD,), jnp.float32),
        pltpu.VMEM((NPAD,), jnp.float32),
        pltpu.VMEM((GSL,), jnp.float32),
        pltpu.VMEM((NB, 128), jnp.float32),
        pltpu.VMEM((NB, 128), jnp.int32),
        pltpu.VMEM_SHARED((NPAD,), jnp.float32),
        pltpu.VMEM_SHARED((NPAD,), jnp.float32),
        pltpu.VMEM_SHARED((NPAD,), jnp.float32),
        pltpu.VMEM_SHARED((GP,), jnp.float32),
        pltpu.SemaphoreType.DMA,
    ],
)
def _k2_scatter(e_hbm, v_hbm, g_out,
                srcv, dstv, dstm, hrows, ones, hv, vslv, dslv, aslv,
                dinvv, av, zbuf,
                wrows, irows, hist_sh, dinv_sh, a_sh, g_sh, sem):
    c = lax.axis_index("c")
    s = lax.axis_index("s")
    tid = c * NS + s
    mtid = (1 - c) * NS + s

    for k in range(128 // L):
        ones[pl.ds(k * L, L)] = jnp.ones((L,), jnp.float32)
    _zero_vmem(zbuf, GSL)
    hoff = pl.multiple_of(s * HSL, 128)
    goff = pl.multiple_of(s * GSL, 128)
    pltpu.sync_copy(zbuf.at[pl.ds(0, HSL)], hist_sh.at[pl.ds(hoff, HSL)])
    pltpu.sync_copy(zbuf, g_sh.at[pl.ds(goff, GSL)])

    nrows, gbs, shift = _tile_rows(tid)
    mnrows, mgbs, mshift = _tile_rows(mtid)
    pltpu.sync_copy(e_hbm.at[0, pl.ds(gbs * 128, CHE * 128)], srcv)
    pltpu.sync_copy(e_hbm.at[1, pl.ds(gbs * 128, CHE * 128)], dstv)
    pltpu.sync_copy(e_hbm.at[1, pl.ds(mgbs * 128, CHE * 128)], dstm)
    _sanitize(srcv, 0, shift)
    _sanitize(srcv, shift + nrows, CHE)
    _sanitize(dstv, 0, shift)
    _sanitize(dstv, shift + nrows, CHE)
    _sanitize(dstm, 0, mshift)
    _sanitize(dstm, mshift + mnrows, CHE)
    plsc.subcore_barrier()

    def hblk(jb, _):
        descs = []
        for t in range(NBH):
            row = jb * NBH + t
            for k in range(128 // L):
                sl = pl.ds(k * L, L)
                hrows[t, sl] = dstv[pl.ds(row * 128 + k * L, L)]
                hrows[NBH + t, sl] = dstm[pl.ds(row * 128 + k * L, L)]
            descs.append(pltpu.async_copy(
                ones, hist_sh.at[hrows.at[t]], sem, add=True))
            descs.append(pltpu.async_copy(
                ones, hist_sh.at[hrows.at[NBH + t]], sem, add=True))
        for dsc in descs:
            dsc.wait()
        return 0

    lax.fori_loop(0, CHE // NBH, hblk, 0)
    plsc.subcore_barrier()

    pltpu.sync_copy(hist_sh.at[pl.ds(hoff, HSL)], hv)
    pltpu.sync_copy(v_hbm.at[pl.ds(hoff, HSL)], vslv)

    def dbody(i, _):
        sl = pl.ds(i * L, L)
        y = _fast_rsqrt(hv[sl] + 1.0)
        dslv[sl] = y
        aslv[sl] = vslv[sl] * y
        return 0

    lax.fori_loop(0, HSL // L, dbody, 0)
    pltpu.sync_copy(dslv, dinv_sh.at[pl.ds(hoff, HSL)])
    pltpu.sync_copy(aslv, a_sh.at[pl.ds(hoff, HSL)])
    plsc.subcore_barrier()
    pltpu.sync_copy(dinv_sh, dinvv)
    pltpu.sync_copy(a_sh, av)

    def _do_row(t, s16, d16):
        for k in range(128 // L):
            sl = pl.ds(k * L, L)
            dis = plsc.load_gather(dinvv, [s16[k]])
            ad = plsc.load_gather(av, [d16[k]])
            wrows[t, sl] = ad * dis
            irows[t, sl] = (d16[k] * MAGIC >> SHIFT) * NPAD + s16[k]
        return pltpu.async_copy(
            wrows.at[t], g_sh.at[irows.at[t]], sem, add=True)

    def blk(jb, _):
        descs = []
        for t in range(NB):
            row = jb * NB + t
            s16 = [srcv[pl.ds(row * 128 + k * L, L)] for k in range(128 // L)]
            d16 = [dstv[pl.ds(row * 128 + k * L, L)] for k in range(128 // L)]
            descs.append(_do_row(t, s16, d16))
        for dsc in descs:
            dsc.wait()
        return 0

    lax.fori_loop(0, CHE // NB, blk, 0)

    iota = lax.iota(jnp.int32, L)
    descs = []
    for t in range(NB):
        n16 = []
        for k in range(128 // L):
            if t < 3:
                n = (tid * 3 + t) * 128 + k * L + iota
                n16.append(jnp.minimum(n, NPAD - 1))
            else:
                n16.append(jnp.full((L,), NPAD - 1, jnp.int32))
        descs.append(_do_row(t, n16, n16))
    for dsc in descs:
        dsc.wait()

    plsc.subcore_barrier()
    ooff = pl.multiple_of(c * GP + s * GSL, 128)
    pltpu.sync_copy(g_sh.at[pl.ds(goff, GSL)], g_out.at[pl.ds(ooff, GSL)])


def _k3_body(g_ref, x_ref, pm_ref, w_ref, b_ref, o_ref):
    gg = g_ref[...].reshape(NC * GROW, NPAD)
    g = (gg[0:B] + gg[GROW:GROW + B])[:, :N]
    gx = jnp.dot(g, x_ref[...], preferred_element_type=jnp.float32)
    r = jnp.dot(gx, w_ref[...], preferred_element_type=jnp.float32)
    msum = jnp.sum(pm_ref[...], axis=1)
    o_ref[...] = r + msum[:, None] * b_ref[...][None, :]


_k3_tail = pl.pallas_call(
    _k3_body,
    out_shape=jax.ShapeDtypeStruct((B, D), jnp.float32),
)


def kernel(x, edge_index, protein_mask, W, b):
    e = edge_index.astype(jnp.int32)
    v = jnp.concatenate(
        [protein_mask.reshape(-1), jnp.zeros((NPAD - N,), jnp.float32)])
    gflat = _k2_scatter(e, v)
    return _k3_tail(gflat, x, protein_mask, W, b)
